# baseline (speedup 1.0000x reference)
import functools

import jax
import jax.numpy as jnp
from jax import lax
from jax.experimental import pallas as pl
from jax.experimental.pallas import tpu as pltpu
from jax.experimental.pallas import tpu_sc as plsc

_f32 = jnp.float32
_i32 = jnp.int32

D = 50
DP = 64
A = 20
HALF = 25
PACKW = 32
ONES_COL = HALF


def _proj_attr_body(x_ref, w_ref, b_ref, o_ref):
    o_ref[...] = (
        jnp.dot(x_ref[...], w_ref[...], preferred_element_type=_f32) + b_ref[...]
    )


def _tc_project_attr(attr_table, wp, bp):
    v = attr_table.shape[0]
    bl = 2000
    return pl.pallas_call(
        _proj_attr_body,
        grid=(v // bl,),
        in_specs=[
            pl.BlockSpec((bl, attr_table.shape[1]), lambda i: (i, 0)),
            pl.BlockSpec((wp.shape[0], DP), lambda i: (0, 0)),
            pl.BlockSpec((1, DP), lambda i: (0, 0)),
        ],
        out_specs=pl.BlockSpec((bl, DP), lambda i: (i, 0)),
        out_shape=jax.ShapeDtypeStruct((v, DP), _f32),
    )(attr_table, wp, bp)


def _rel_body(r_ref, wp_ref, bp_ref, waug_ref, oproj_ref, oaug_ref):
    r = r_ref[...]
    oproj_ref[...] = jnp.dot(r, wp_ref[...], preferred_element_type=_f32) + bp_ref[...]
    oaug_ref[...] = jnp.dot(r, waug_ref[...], preferred_element_type=_f32)


def _tc_rel_tables(rel_table, wp, bp, w_mid_aug):
    nr = rel_table.shape[0]
    return pl.pallas_call(
        _rel_body,
        out_shape=(
            jax.ShapeDtypeStruct((nr, DP), _f32),
            jax.ShapeDtypeStruct((nr, DP), _f32),
        ),
    )(rel_table, wp, bp, w_mid_aug)


def _pre_finish(h, ps_ref, u_ref, p0_ref, p1_ref, psd_ref, h_ref, s2_ref,
                pack_ref, sd_ref):
    h_ref[...] = h
    s2_ref[...] = jnp.dot(h, ps_ref[...], preferred_element_type=_f32)
    pack_ref[0] = jnp.dot(h, p0_ref[...], preferred_element_type=_f32) + u_ref[...]
    pack_ref[1] = jnp.dot(h, p1_ref[...], preferred_element_type=_f32) + u_ref[...]
    sd_ref[...] = jnp.dot(h, psd_ref[...], preferred_element_type=_f32)


def _pre_body(x_ref, w_ref, p0_ref, p1_ref, ps_ref, u_ref, psd_ref, h_ref,
              s2_ref, pack_ref, sd_ref):
    h = jnp.dot(x_ref[...], w_ref[...], preferred_element_type=_f32)
    _pre_finish(h, ps_ref, u_ref, p0_ref, p1_ref, psd_ref, h_ref, s2_ref,
                pack_ref, sd_ref)


def _pre_body_rel(x_ref, xr_ref, w_ref, p0_ref, p1_ref, ps_ref, u_ref, psd_ref,
                  h_ref, s2_ref, pack_ref, sd_ref):
    h = jnp.dot(x_ref[...], w_ref[...], preferred_element_type=_f32) + xr_ref[...]
    _pre_finish(h, ps_ref, u_ref, p0_ref, p1_ref, psd_ref, h_ref, s2_ref,
                pack_ref, sd_ref)


def _tc_pre(x, w_aug, p0, p1, ps, u_row, psd, x_rel=None):
    n = x.shape[0]
    xw = x.shape[1]
    bl = 1000
    xspec = [pl.BlockSpec((bl, xw), lambda i: (i, 0))]
    args = [x]
    body = _pre_body
    if x_rel is not None:
        xspec.append(pl.BlockSpec((bl, DP), lambda i: (i, 0)))
        args.append(x_rel)
        body = _pre_body_rel
    return pl.pallas_call(
        body,
        grid=(n // bl,),
        in_specs=xspec + [
            pl.BlockSpec((xw, DP), lambda i: (0, 0)),
            pl.BlockSpec((DP, PACKW), lambda i: (0, 0)),
            pl.BlockSpec((DP, PACKW), lambda i: (0, 0)),
            pl.BlockSpec((DP, 2), lambda i: (0, 0)),
            pl.BlockSpec((1, PACKW), lambda i: (0, 0)),
            pl.BlockSpec((DP, 16), lambda i: (0, 0)),
        ],
        out_specs=(
            pl.BlockSpec((bl, DP), lambda i: (i, 0)),
            pl.BlockSpec((bl, 2), lambda i: (i, 0)),
            pl.BlockSpec((2, bl, PACKW), lambda i: (0, i, 0)),
            pl.BlockSpec((bl, 16), lambda i: (i, 0)),
        ),
        out_shape=(
            jax.ShapeDtypeStruct((n, DP), _f32),
            jax.ShapeDtypeStruct((n, 2), _f32),
            jax.ShapeDtypeStruct((2, n, PACKW), _f32),
            jax.ShapeDtypeStruct((n, 16), _f32),
        ),
    )(*args, w_aug, p0, p1, ps, u_row, psd)


def _combine_body(acc_ref, h_ref, m_ref, b_ref, q0_ref, q1_ref, qd_ref, s_ref,
                  bb_ref, o_ref):
    a0 = acc_ref[0]
    a1 = acc_ref[1]
    h = h_ref[...]
    num = (jnp.dot(a0, q0_ref[...], preferred_element_type=_f32)
           + jnp.dot(a1, q1_ref[...], preferred_element_type=_f32))
    den = jnp.dot(a0, qd_ref[...], preferred_element_type=_f32)
    h_sel = jnp.dot(h, s_ref[...], preferred_element_type=_f32)
    ssum = jnp.dot(h, bb_ref[...], preferred_element_type=_f32)
    ssum = jnp.where(ssum >= 0.0, ssum, 0.2 * ssum)
    el = jnp.exp(ssum - m_ref[...])
    o_ref[...] = (num + el * h_sel) / (den + el + 1e-16) + b_ref[...]


def _tc_combine(acc, h_aug, m_row, bias_row, q0, q1, qd, s_mat, b_mat):
    n = h_aug.shape[0]
    bl = 1000
    return pl.pallas_call(
        _combine_body,
        grid=(n // bl,),
        in_specs=[
            pl.BlockSpec((2, bl, PACKW), lambda i: (0, i, 0)),
            pl.BlockSpec((bl, DP), lambda i: (i, 0)),
            pl.BlockSpec((1, DP), lambda i: (0, 0)),
            pl.BlockSpec((1, DP), lambda i: (0, 0)),
            pl.BlockSpec((PACKW, DP), lambda i: (0, 0)),
            pl.BlockSpec((PACKW, DP), lambda i: (0, 0)),
            pl.BlockSpec((PACKW, DP), lambda i: (0, 0)),
            pl.BlockSpec((DP, DP), lambda i: (0, 0)),
            pl.BlockSpec((DP, DP), lambda i: (0, 0)),
        ],
        out_specs=pl.BlockSpec((bl, DP), lambda i: (i, 0)),
        out_shape=jax.ShapeDtypeStruct((n, DP), _f32),
    )(acc, h_aug, m_row, bias_row, q0, q1, qd, s_mat, b_mat)


_XW = 128


def _stage_e_body(proj_attr, proj_rel, rel_aug, hidx_hbm, tidx_hbm, ridx_hbm,
                  x_hbm, xrel_hbm,
                  hib, tib, rib, ah2, at2, relp2, relaug2, xrow,
                  semi0, semi1, semg0, semg1):
    c = lax.axis_index("c")
    s = lax.axis_index("s")
    w = s * 2 + c
    iota = lax.broadcasted_iota(_i32, (16,), 0)
    c20 = iota * A
    zero16 = jnp.zeros((16,), _f32)
    semi = [semi0, semi1]
    semg = [semg0, semg1]

    for r in range(16):
        for k in range(_XW // 16):
            xrow[r, pl.ds(k * 16, 16)] = zero16

    nblocks = 3125
    count = (nblocks - w + 31) // 32

    def node0_of(g):
        return (w + g * 32) * 16

    def fire_idx(g, b):
        n0 = node0_of(g)
        pltpu.async_copy(hidx_hbm.at[pl.ds(n0 * A, 16 * A)], hib.at[b],
                         semi[b])
        pltpu.async_copy(tidx_hbm.at[pl.ds(n0 * A, 16 * A)], tib.at[b],
                         semi[b])
        pltpu.async_copy(ridx_hbm.at[pl.ds(n0, 16)], rib.at[b], semi[b])

    def wait_idx(b):
        pltpu.make_async_copy(hidx_hbm.at[pl.ds(0, 16 * A)], hib.at[b],
                              semi[b]).wait()
        pltpu.make_async_copy(tidx_hbm.at[pl.ds(0, 16 * A)], tib.at[b],
                              semi[b]).wait()
        pltpu.make_async_copy(ridx_hbm.at[pl.ds(0, 16)], rib.at[b],
                              semi[b]).wait()

    def fire_gathers(b):
        for k in range(4):
            pltpu.async_copy(proj_attr.at[hib.at[b, pl.ds(k * 80, 80)]],
                             ah2.at[b, pl.ds(k * 80, 80)], semg[b])
            pltpu.async_copy(proj_attr.at[tib.at[b, pl.ds(k * 80, 80)]],
                             at2.at[b, pl.ds(k * 80, 80)], semg[b])
        pltpu.async_copy(proj_rel.at[rib.at[b]], relp2.at[b], semg[b])
        pltpu.async_copy(rel_aug.at[rib.at[b]], relaug2.at[b], semg[b])

    def wait_gathers(b):
        for k in range(4):
            pltpu.make_async_copy(proj_attr.at[hib.at[b, pl.ds(k * 80, 80)]],
                                  ah2.at[b, pl.ds(k * 80, 80)], semg[b]).wait()
            pltpu.make_async_copy(proj_attr.at[tib.at[b, pl.ds(k * 80, 80)]],
                                  at2.at[b, pl.ds(k * 80, 80)], semg[b]).wait()
        pltpu.make_async_copy(proj_rel.at[rib.at[b]], relp2.at[b],
                              semg[b]).wait()
        pltpu.make_async_copy(rel_aug.at[rib.at[b]], relaug2.at[b],
                              semg[b]).wait()

    def attend(av_ref, rp_ref, off):
        def logits_step(dd, accs):
            dcol = jnp.full((16,), 0, _i32) + dd
            rv = plsc.load_gather(rp_ref, [iota, dcol])
            return tuple(
                accs[a] + rv * plsc.load_gather(av_ref, [c20 + a, dcol])
                for a in range(A)
            )

        logits = lax.fori_loop(
            0, D, logits_step, tuple(zero16 for _ in range(A)))
        m = logits[0]
        for a in range(1, A):
            m = jnp.maximum(m, logits[a])
        es = [jnp.exp(logits[a] - m) for a in range(A)]
        tot = es[0]
        for a in range(1, A):
            tot = tot + es[a]
        inv = 1.0 / tot
        attn = [e * inv for e in es]

        def wsum_step(dd, _):
            dcol = jnp.full((16,), 0, _i32) + dd
            hv = attn[0] * plsc.load_gather(av_ref, [c20, dcol])
            for a in range(1, A):
                hv = hv + attn[a] * plsc.load_gather(av_ref, [c20 + a, dcol])
            plsc.store_scatter(xrow, [iota, dcol + off], hv)
            return 0

        lax.fori_loop(0, D, wsum_step, 0)

    fire_idx(0, 0)
    wait_idx(0)
    fire_gathers(0)
    fire_idx(1, 1)

    def pair(g2, _):
        for b in range(2):
            nb = 1 - b
            g = g2 * 2 + b

            @pl.when(g < count)
            def _():
                @pl.when(g + 1 < count)
                def _():
                    wait_idx(nb)
                    fire_gathers(nb)
                wait_gathers(b)
                attend(ah2.at[b], relp2.at[b], 0)
                attend(at2.at[b], relp2.at[b], DP)
                n0 = node0_of(g)
                pltpu.sync_copy(xrow, x_hbm.at[pl.ds(n0, 16)])
                pltpu.sync_copy(relaug2.at[b], xrel_hbm.at[pl.ds(n0, 16)])

                @pl.when(g + 2 < count)
                def _():
                    fire_idx(g + 2, b)
        return 0

    lax.fori_loop(0, (count + 1) // 2, pair, 0)


def _sc_stage_e(proj_attr, proj_rel, rel_aug, hidx, tidx, ridx):
    n = ridx.shape[0]
    mesh = plsc.VectorSubcoreMesh(core_axis_name="c", subcore_axis_name="s")
    kern = functools.partial(
        pl.kernel,
        mesh=mesh,
        out_type=(
            jax.ShapeDtypeStruct((n, _XW), _f32),
            jax.ShapeDtypeStruct((n, DP), _f32),
        ),
        scratch_types=[
            pltpu.MemorySpace.VMEM((2, 320), _i32),
            pltpu.MemorySpace.VMEM((2, 320), _i32),
            pltpu.MemorySpace.VMEM((2, 16), _i32),
            pltpu.MemorySpace.VMEM((2, 320, DP), _f32),
            pltpu.MemorySpace.VMEM((2, 320, DP), _f32),
            pltpu.MemorySpace.VMEM((2, 16, DP), _f32),
            pltpu.MemorySpace.VMEM((2, 16, DP), _f32),
            pltpu.MemorySpace.VMEM((16, _XW), _f32),
            pltpu.SemaphoreType.DMA,
            pltpu.SemaphoreType.DMA,
            pltpu.SemaphoreType.DMA,
            pltpu.SemaphoreType.DMA,
        ],
        compiler_params=pltpu.CompilerParams(needs_layout_passes=False, use_tc_tiling_on_sc=False),
    )(_stage_e_body)
    return kern(proj_attr, proj_rel, rel_aug, hidx, tidx, ridx)


SSRC_LANE = 10


_EK = 80


def _edge_body(pack_hbm, sd_hbm, edges_hbm, m_hbm, zeros_hbm, out_hbm,
               ebuf, srcadj, rows2, sdrows2, scaled2, dbuf, m_v,
               semi0, semi1, semg0, semg1, sems0, sems1, acc_sh):
    c = lax.axis_index("c")
    s = lax.axis_index("s")
    n = sd_hbm.shape[0]
    rows_per_tile = n // 16
    coff = c * n
    k = _EK
    iota = lax.broadcasted_iota(_i32, (16,), 0)

    pltpu.sync_copy(m_hbm, m_v)
    pltpu.sync_copy(zeros_hbm.at[pl.ds(s * rows_per_tile, rows_per_tile)],
                    acc_sh.at[pl.ds(s * rows_per_tile, rows_per_tile)])
    plsc.subcore_barrier()

    e_total = edges_hbm.shape[1]
    per_tile = e_total // 16
    nchunks = per_tile // k
    base0 = s * per_tile
    semi = [semi0, semi1]
    semg = [semg0, semg1]
    sems = [sems0, sems1]
    mv = m_v[...]

    def fire_idx(g, b):
        pltpu.async_copy(edges_hbm.at[:, pl.ds(base0 + g * k, k)],
                         ebuf.at[b], semi[b])

    def fire_gathers(g, b):
        for j in range(k // 16):
            sv = ebuf[b, 0, pl.ds(j * 16, 16)]
            srcadj[b, pl.ds(j * 16, 16)] = sv + coff
        pltpu.async_copy(pack_hbm.at[srcadj.at[b]], rows2.at[b], semg[b])
        pltpu.async_copy(sd_hbm.at[ebuf.at[b, 1]], sdrows2.at[b], semg[b])

    def consume(g, b):
        for j in range(k // 16):
            ridx = iota + j * 16
            ss = plsc.load_gather(rows2.at[b], [ridx, jnp.full((16,), 26, _i32)])
            sd = plsc.load_gather(sdrows2.at[b], [ridx, jnp.full((16,), 0, _i32)])
            al = ss + sd
            al = jnp.where(al >= 0.0, al, 0.2 * al)
            ev = jnp.exp(al - mv)
            dbuf[b, pl.ds(j * 16, 16)] = ebuf[b, 1, pl.ds(j * 16, 16)]
            for i in range(16):
                row = j * 16 + i
                es = jnp.take(ev, jnp.full((16,), i, _i32))
                scaled2[b, row, pl.ds(0, 16)] = rows2[b, row, pl.ds(0, 16)] * es
                scaled2[b, row, pl.ds(16, 16)] = rows2[b, row, pl.ds(16, 16)] * es
        pltpu.async_copy(scaled2.at[b], acc_sh.at[dbuf.at[b]], sems[b],
                         add=True)

    def drain_scatter(b):
        pltpu.make_async_copy(scaled2.at[b], acc_sh.at[dbuf.at[b]],
                              sems[b]).wait()

    fire_idx(0, 0)
    pltpu.make_async_copy(edges_hbm.at[:, pl.ds(0, k)], ebuf.at[0],
                          semi[0]).wait()
    fire_gathers(0, 0)
    fire_idx(1, 1)

    def pair(g2, _):
        for b in range(2):
            nb = 1 - b
            g = g2 * 2 + b

            @pl.when(g + 1 < nchunks)
            def _():
                pltpu.make_async_copy(
                    edges_hbm.at[:, pl.ds(0, k)], ebuf.at[nb], semi[nb]).wait()
                fire_gathers(g + 1, nb)

            @pl.when(g >= 2)
            def _():
                drain_scatter(b)
            pltpu.make_async_copy(pack_hbm.at[srcadj.at[b]], rows2.at[b],
                                  semg[b]).wait()
            pltpu.make_async_copy(sd_hbm.at[ebuf.at[b, 1]], sdrows2.at[b],
                                  semg[b]).wait()
            consume(g, b)

            @pl.when(g + 2 < nchunks)
            def _():
                fire_idx(g + 2, b)
        return 0

    lax.fori_loop(0, nchunks // 2, pair, 0)
    drain_scatter(0)
    drain_scatter(1)
    plsc.subcore_barrier()
    pltpu.sync_copy(acc_sh.at[pl.ds(s * rows_per_tile, rows_per_tile)],
                    out_hbm.at[pl.ds(coff + s * rows_per_tile, rows_per_tile)])


def _sc_edge_pass(pack_flat, sd16, edge_index, m_vec, zeros_nk):
    n = sd16.shape[0]
    mesh = plsc.VectorSubcoreMesh(core_axis_name="c", subcore_axis_name="s")
    kern = functools.partial(
        pl.kernel,
        mesh=mesh,
        out_type=jax.ShapeDtypeStruct((2 * n, PACKW), _f32),
        scratch_types=[
            pltpu.MemorySpace.VMEM((2, 2, _EK), _i32),
            pltpu.MemorySpace.VMEM((2, _EK), _i32),
            pltpu.MemorySpace.VMEM((2, _EK, PACKW), _f32),
            pltpu.MemorySpace.VMEM((2, _EK, 16), _f32),
            pltpu.MemorySpace.VMEM((2, _EK, PACKW), _f32),
            pltpu.MemorySpace.VMEM((2, _EK), _i32),
            pltpu.MemorySpace.VMEM((16,), _f32),
            pltpu.SemaphoreType.DMA,
            pltpu.SemaphoreType.DMA,
            pltpu.SemaphoreType.DMA,
            pltpu.SemaphoreType.DMA,
            pltpu.SemaphoreType.DMA,
            pltpu.SemaphoreType.DMA,
            pltpu.MemorySpace.VMEM_SHARED((n, PACKW), _f32),
        ],
        compiler_params=pltpu.CompilerParams(needs_layout_passes=False, use_tc_tiling_on_sc=False),
    )(_edge_body)
    return kern(pack_flat, sd16, edge_index, m_vec, zeros_nk)


def _aug_w(w_part, a_src, a_dst):
    k = w_part.shape[0]
    return jnp.concatenate(
        [w_part, (w_part @ a_src)[:, None], (w_part @ a_dst)[:, None],
         jnp.zeros((k, DP - D - 2), _f32)], axis=1)


def kernel(h_attributes, r_idx, t_attributes, edge_index, attr_table, rel_table,
           femb_W, femb_b, gat1_W, gat1_att_src, gat1_att_dst, gat1_bias,
           gat2_W, gat2_att_src, gat2_att_dst, gat2_bias):
    n = h_attributes.shape[0]

    wp = jnp.concatenate([femb_W, jnp.zeros((D, DP - D), _f32)], axis=1)
    bp = jnp.concatenate([femb_b, jnp.zeros((DP - D,), _f32)])[None, :]
    w1_mid_aug = _aug_w(gat1_W[D:2 * D], gat1_att_src, gat1_att_dst)
    w_cat1 = jnp.zeros((_XW, DP), _f32)
    w_cat1 = w_cat1.at[0:D].set(_aug_w(gat1_W[0:D], gat1_att_src, gat1_att_dst))
    w_cat1 = w_cat1.at[DP:DP + D].set(
        _aug_w(gat1_W[2 * D:3 * D], gat1_att_src, gat1_att_dst))
    w2_aug = jnp.zeros((DP, DP), _f32)
    w2_aug = w2_aug.at[0:D].set(_aug_w(gat2_W, gat2_att_src, gat2_att_dst))

    eye = jnp.eye(DP, dtype=_f32)
    p0 = jnp.zeros((DP, PACKW), _f32).at[0:HALF, 0:HALF].set(jnp.eye(HALF, dtype=_f32))
    p1 = jnp.zeros((DP, PACKW), _f32).at[HALF:2 * HALF, 0:HALF].set(
        jnp.eye(HALF, dtype=_f32))
    p0 = p0.at[D, 26].set(1.0)
    p1 = p1.at[D, 26].set(1.0)
    psd = jnp.zeros((DP, 16), _f32).at[D + 1, 0].set(1.0)
    u_row = jnp.zeros((1, PACKW), _f32).at[0, ONES_COL].set(1.0)
    ps = jnp.zeros((DP, 2), _f32).at[D, 0].set(1.0).at[D + 1, 1].set(1.0)
    q0 = jnp.zeros((PACKW, DP), _f32).at[0:HALF, 0:HALF].set(jnp.eye(HALF, dtype=_f32))
    q1 = jnp.zeros((PACKW, DP), _f32).at[0:HALF, HALF:2 * HALF].set(
        jnp.eye(HALF, dtype=_f32))
    qd = jnp.zeros((PACKW, DP), _f32).at[ONES_COL, :].set(1.0)
    s_mat = eye.at[D:, :].set(0.0)
    b_mat = jnp.zeros((DP, DP), _f32).at[D, :].set(1.0).at[D + 1, :].set(1.0)
    bias1_row = jnp.concatenate([gat1_bias, jnp.zeros((DP - D,), _f32)])[None, :]
    bias2_row = jnp.concatenate([gat2_bias, jnp.zeros((DP - D,), _f32)])[None, :]

    hidx = h_attributes.reshape(-1).astype(_i32)
    tidx = t_attributes.reshape(-1).astype(_i32)
    ridx = r_idx.astype(_i32)
    edges = edge_index.astype(_i32)
    zeros_nk = jnp.zeros((n, PACKW), _f32)

    proj_attr = _tc_project_attr(attr_table, wp, bp)
    proj_rel, rel_aug1 = _tc_rel_tables(rel_table, wp, bp, w1_mid_aug)

    x_sc, x_rel = _sc_stage_e(proj_attr, proj_rel, rel_aug1, hidx, tidx, ridx)

    def gat_layer(x, w_aug, bias_row, x_rel=None):
        h_aug, s2, pack, sd16 = _tc_pre(x, w_aug, p0, p1, ps, u_row, psd, x_rel)
        mx = jnp.max(s2, axis=0)
        m = mx[0] + mx[1]
        m = jnp.where(m >= 0.0, m, 0.2 * m)
        m_vec = jnp.full((16,), m, _f32)
        m_row = jnp.full((1, DP), m, _f32)
        acc = _sc_edge_pass(pack.reshape(2 * n, PACKW), sd16, edges,
                            m_vec, zeros_nk)
        return _tc_combine(acc.reshape(2, n, PACKW), h_aug, m_row, bias_row,
                           q0, q1, qd, s_mat, b_mat)

    x1 = gat_layer(x_sc, w_cat1, bias1_row, x_rel)
    x2 = gat_layer(x1, w2_aug, bias2_row)
    return x2[:, :D]


# speedup vs baseline: 2.0195x; 2.0195x over previous
import functools

import jax
import jax.numpy as jnp
from jax import lax
from jax.experimental import pallas as pl
from jax.experimental.pallas import tpu as pltpu
from jax.experimental.pallas import tpu_sc as plsc

_f32 = jnp.float32
_i32 = jnp.int32

D = 50
DP = 64
A = 20
HALF = 25
PACKW = 32
ONES_COL = HALF


def _proj_attr_body(x_ref, w_ref, b_ref, o_ref):
    o_ref[...] = (
        jnp.dot(x_ref[...], w_ref[...], preferred_element_type=_f32) + b_ref[...]
    )


def _tc_project_attr(attr_table, wp, bp):
    v = attr_table.shape[0]
    bl = 2000
    return pl.pallas_call(
        _proj_attr_body,
        grid=(v // bl,),
        in_specs=[
            pl.BlockSpec((bl, attr_table.shape[1]), lambda i: (i, 0)),
            pl.BlockSpec((wp.shape[0], DP), lambda i: (0, 0)),
            pl.BlockSpec((1, DP), lambda i: (0, 0)),
        ],
        out_specs=pl.BlockSpec((bl, DP), lambda i: (i, 0)),
        out_shape=jax.ShapeDtypeStruct((v, DP), _f32),
    )(attr_table, wp, bp)


def _rel_body(r_ref, wp_ref, bp_ref, waug_ref, oproj_ref, oaug_ref):
    r = r_ref[...]
    oproj_ref[...] = jnp.dot(r, wp_ref[...], preferred_element_type=_f32) + bp_ref[...]
    oaug_ref[...] = jnp.dot(r, waug_ref[...], preferred_element_type=_f32)


def _tc_rel_tables(rel_table, wp, bp, w_mid_aug):
    nr = rel_table.shape[0]
    return pl.pallas_call(
        _rel_body,
        out_shape=(
            jax.ShapeDtypeStruct((nr, DP), _f32),
            jax.ShapeDtypeStruct((nr, DP), _f32),
        ),
    )(rel_table, wp, bp, w_mid_aug)


def _pre_finish(h, ps_ref, u_ref, p0_ref, p1_ref, psd_ref, h_ref, s2_ref,
                pack_ref, sd_ref):
    h_ref[...] = h
    s2_ref[...] = jnp.dot(h, ps_ref[...], preferred_element_type=_f32)
    pack_ref[0] = jnp.dot(h, p0_ref[...], preferred_element_type=_f32) + u_ref[...]
    pack_ref[1] = jnp.dot(h, p1_ref[...], preferred_element_type=_f32) + u_ref[...]
    sd_ref[...] = jnp.dot(h, psd_ref[...], preferred_element_type=_f32)


def _pre_body(x_ref, w_ref, p0_ref, p1_ref, ps_ref, u_ref, psd_ref, h_ref,
              s2_ref, pack_ref, sd_ref):
    h = jnp.dot(x_ref[...], w_ref[...], preferred_element_type=_f32)
    _pre_finish(h, ps_ref, u_ref, p0_ref, p1_ref, psd_ref, h_ref, s2_ref,
                pack_ref, sd_ref)


def _pre_body_rel(x_ref, xr_ref, w_ref, p0_ref, p1_ref, ps_ref, u_ref, psd_ref,
                  h_ref, s2_ref, pack_ref, sd_ref):
    h = jnp.dot(x_ref[...], w_ref[...], preferred_element_type=_f32) + xr_ref[...]
    _pre_finish(h, ps_ref, u_ref, p0_ref, p1_ref, psd_ref, h_ref, s2_ref,
                pack_ref, sd_ref)


def _tc_pre(x, w_aug, p0, p1, ps, u_row, psd, x_rel=None):
    n = x.shape[0]
    xw = x.shape[1]
    bl = 1000
    xspec = [pl.BlockSpec((bl, xw), lambda i: (i, 0))]
    args = [x]
    body = _pre_body
    if x_rel is not None:
        xspec.append(pl.BlockSpec((bl, DP), lambda i: (i, 0)))
        args.append(x_rel)
        body = _pre_body_rel
    return pl.pallas_call(
        body,
        grid=(n // bl,),
        in_specs=xspec + [
            pl.BlockSpec((xw, DP), lambda i: (0, 0)),
            pl.BlockSpec((DP, PACKW), lambda i: (0, 0)),
            pl.BlockSpec((DP, PACKW), lambda i: (0, 0)),
            pl.BlockSpec((DP, 2), lambda i: (0, 0)),
            pl.BlockSpec((1, PACKW), lambda i: (0, 0)),
            pl.BlockSpec((DP, 16), lambda i: (0, 0)),
        ],
        out_specs=(
            pl.BlockSpec((bl, DP), lambda i: (i, 0)),
            pl.BlockSpec((bl, 2), lambda i: (i, 0)),
            pl.BlockSpec((2, bl, PACKW), lambda i: (0, i, 0)),
            pl.BlockSpec((bl, 16), lambda i: (i, 0)),
        ),
        out_shape=(
            jax.ShapeDtypeStruct((n, DP), _f32),
            jax.ShapeDtypeStruct((n, 2), _f32),
            jax.ShapeDtypeStruct((2, n, PACKW), _f32),
            jax.ShapeDtypeStruct((n, 16), _f32),
        ),
    )(*args, w_aug, p0, p1, ps, u_row, psd)


def _combine_body(acc_ref, h_ref, m_ref, b_ref, q0_ref, q1_ref, qd_ref, s_ref,
                  bb_ref, o_ref):
    a0 = acc_ref[0]
    a1 = acc_ref[1]
    h = h_ref[...]
    num = (jnp.dot(a0, q0_ref[...], preferred_element_type=_f32)
           + jnp.dot(a1, q1_ref[...], preferred_element_type=_f32))
    den = jnp.dot(a0, qd_ref[...], preferred_element_type=_f32)
    h_sel = jnp.dot(h, s_ref[...], preferred_element_type=_f32)
    ssum = jnp.dot(h, bb_ref[...], preferred_element_type=_f32)
    ssum = jnp.where(ssum >= 0.0, ssum, 0.2 * ssum)
    el = jnp.exp(ssum - m_ref[...])
    o_ref[...] = (num + el * h_sel) / (den + el + 1e-16) + b_ref[...]


def _tc_combine(acc, h_aug, m_row, bias_row, q0, q1, qd, s_mat, b_mat):
    n = h_aug.shape[0]
    bl = 1000
    return pl.pallas_call(
        _combine_body,
        grid=(n // bl,),
        in_specs=[
            pl.BlockSpec((2, bl, PACKW), lambda i: (0, i, 0)),
            pl.BlockSpec((bl, DP), lambda i: (i, 0)),
            pl.BlockSpec((1, DP), lambda i: (0, 0)),
            pl.BlockSpec((1, DP), lambda i: (0, 0)),
            pl.BlockSpec((PACKW, DP), lambda i: (0, 0)),
            pl.BlockSpec((PACKW, DP), lambda i: (0, 0)),
            pl.BlockSpec((PACKW, DP), lambda i: (0, 0)),
            pl.BlockSpec((DP, DP), lambda i: (0, 0)),
            pl.BlockSpec((DP, DP), lambda i: (0, 0)),
        ],
        out_specs=pl.BlockSpec((bl, DP), lambda i: (i, 0)),
        out_shape=jax.ShapeDtypeStruct((n, DP), _f32),
    )(acc, h_aug, m_row, bias_row, q0, q1, qd, s_mat, b_mat)


_XW = 128


def _stage_e_body(proj_attr, proj_rel, rel_aug, hidx_hbm, tidx_hbm, ridx_hbm,
                  x_hbm, xrel_hbm,
                  hib, tib, rib, ah2, at2, relp2, relaug2, xrow,
                  semi0, semi1, semg0, semg1):
    c = lax.axis_index("c")
    s = lax.axis_index("s")
    w = s * 2 + c
    iota = lax.broadcasted_iota(_i32, (16,), 0)
    c20 = iota * A
    zero16 = jnp.zeros((16,), _f32)
    semi = [semi0, semi1]
    semg = [semg0, semg1]

    for r in range(16):
        for k in range(_XW // 16):
            xrow[r, pl.ds(k * 16, 16)] = zero16

    nblocks = 3125
    count = (nblocks - w + 31) // 32

    def node0_of(g):
        return (w + g * 32) * 16

    def fire_idx(g, b):
        n0 = node0_of(g)
        pltpu.async_copy(hidx_hbm.at[pl.ds(n0 * A, 16 * A)], hib.at[b],
                         semi[b])
        pltpu.async_copy(tidx_hbm.at[pl.ds(n0 * A, 16 * A)], tib.at[b],
                         semi[b])
        pltpu.async_copy(ridx_hbm.at[pl.ds(n0, 16)], rib.at[b], semi[b])

    def wait_idx(b):
        pltpu.make_async_copy(hidx_hbm.at[pl.ds(0, 16 * A)], hib.at[b],
                              semi[b]).wait()
        pltpu.make_async_copy(tidx_hbm.at[pl.ds(0, 16 * A)], tib.at[b],
                              semi[b]).wait()
        pltpu.make_async_copy(ridx_hbm.at[pl.ds(0, 16)], rib.at[b],
                              semi[b]).wait()

    def fire_gathers(b):
        for k in range(4):
            pltpu.async_copy(proj_attr.at[hib.at[b, pl.ds(k * 80, 80)]],
                             ah2.at[b, pl.ds(k * 80, 80)], semg[b])
            pltpu.async_copy(proj_attr.at[tib.at[b, pl.ds(k * 80, 80)]],
                             at2.at[b, pl.ds(k * 80, 80)], semg[b])
        pltpu.async_copy(proj_rel.at[rib.at[b]], relp2.at[b], semg[b])
        pltpu.async_copy(rel_aug.at[rib.at[b]], relaug2.at[b], semg[b])

    def wait_gathers(b):
        for k in range(4):
            pltpu.make_async_copy(proj_attr.at[hib.at[b, pl.ds(k * 80, 80)]],
                                  ah2.at[b, pl.ds(k * 80, 80)], semg[b]).wait()
            pltpu.make_async_copy(proj_attr.at[tib.at[b, pl.ds(k * 80, 80)]],
                                  at2.at[b, pl.ds(k * 80, 80)], semg[b]).wait()
        pltpu.make_async_copy(proj_rel.at[rib.at[b]], relp2.at[b],
                              semg[b]).wait()
        pltpu.make_async_copy(rel_aug.at[rib.at[b]], relaug2.at[b],
                              semg[b]).wait()

    neg_big = jnp.full((16,), -1e30, _f32)
    nchunk = DP // 16

    def attend(av_ref, rp_ref, off, j):
        rv = [rp_ref[j, pl.ds(kk * 16, 16)] for kk in range(nchunk)]
        lv0 = neg_big
        lv1 = neg_big
        for a in range(A):
            row = j * A + a
            part = av_ref[row, pl.ds(0, 16)] * rv[0]
            for kk in range(1, nchunk):
                part = part + av_ref[row, pl.ds(kk * 16, 16)] * rv[kk]
            tot = jnp.sum(part)
            if a < 16:
                lv0 = jnp.where(iota == a, tot, lv0)
            else:
                lv1 = jnp.where(iota == (a - 16), tot, lv1)
        m = jnp.maximum(jnp.max(lv0), jnp.max(lv1))
        e0 = jnp.exp(lv0 - m)
        e1 = jnp.exp(lv1 - m)
        tot = jnp.sum(e0) + jnp.sum(e1)
        attn0 = e0 / tot
        attn1 = e1 / tot
        hv = [zero16 for _ in range(nchunk)]
        for a in range(A):
            row = j * A + a
            aa = attn0[a] if a < 16 else attn1[a - 16]
            for kk in range(nchunk):
                hv[kk] = hv[kk] + av_ref[row, pl.ds(kk * 16, 16)] * aa
        for kk in range(nchunk):
            xrow[j, pl.ds(off + kk * 16, 16)] = hv[kk]

    fire_idx(0, 0)
    wait_idx(0)
    fire_gathers(0)
    fire_idx(1, 1)

    def pair(g2, _):
        for b in range(2):
            nb = 1 - b
            g = g2 * 2 + b

            @pl.when(g < count)
            def _():
                @pl.when(g + 1 < count)
                def _():
                    wait_idx(nb)
                    fire_gathers(nb)
                wait_gathers(b)

                def node_step(j, _):
                    attend(ah2.at[b], relp2.at[b], 0, j)
                    attend(at2.at[b], relp2.at[b], DP, j)
                    return 0

                lax.fori_loop(0, 16, node_step, 0)
                n0 = node0_of(g)
                pltpu.sync_copy(xrow, x_hbm.at[pl.ds(n0, 16)])
                pltpu.sync_copy(relaug2.at[b], xrel_hbm.at[pl.ds(n0, 16)])

                @pl.when(g + 2 < count)
                def _():
                    fire_idx(g + 2, b)
        return 0

    lax.fori_loop(0, (count + 1) // 2, pair, 0)


def _sc_stage_e(proj_attr, proj_rel, rel_aug, hidx, tidx, ridx):
    n = ridx.shape[0]
    mesh = plsc.VectorSubcoreMesh(core_axis_name="c", subcore_axis_name="s")
    kern = functools.partial(
        pl.kernel,
        mesh=mesh,
        out_type=(
            jax.ShapeDtypeStruct((n, _XW), _f32),
            jax.ShapeDtypeStruct((n, DP), _f32),
        ),
        scratch_types=[
            pltpu.MemorySpace.VMEM((2, 320), _i32),
            pltpu.MemorySpace.VMEM((2, 320), _i32),
            pltpu.MemorySpace.VMEM((2, 16), _i32),
            pltpu.MemorySpace.VMEM((2, 320, DP), _f32),
            pltpu.MemorySpace.VMEM((2, 320, DP), _f32),
            pltpu.MemorySpace.VMEM((2, 16, DP), _f32),
            pltpu.MemorySpace.VMEM((2, 16, DP), _f32),
            pltpu.MemorySpace.VMEM((16, _XW), _f32),
            pltpu.SemaphoreType.DMA,
            pltpu.SemaphoreType.DMA,
            pltpu.SemaphoreType.DMA,
            pltpu.SemaphoreType.DMA,
        ],
        compiler_params=pltpu.CompilerParams(needs_layout_passes=False, use_tc_tiling_on_sc=False),
    )(_stage_e_body)
    return kern(proj_attr, proj_rel, rel_aug, hidx, tidx, ridx)


SSRC_LANE = 10


_EK = 80


def _edge_body(pack_hbm, sd_hbm, edges_hbm, m_hbm, zeros_hbm, out_hbm,
               ebuf, srcadj, rows2, sdrows2, scaled2, dbuf, m_v,
               semi0, semi1, semg0, semg1, sems0, sems1, acc_sh):
    c = lax.axis_index("c")
    s = lax.axis_index("s")
    n = sd_hbm.shape[0]
    rows_per_tile = n // 16
    coff = c * n
    k = _EK
    iota = lax.broadcasted_iota(_i32, (16,), 0)

    pltpu.sync_copy(m_hbm, m_v)
    pltpu.sync_copy(zeros_hbm.at[pl.ds(s * rows_per_tile, rows_per_tile)],
                    acc_sh.at[pl.ds(s * rows_per_tile, rows_per_tile)])
    plsc.subcore_barrier()

    e_total = edges_hbm.shape[1]
    per_tile = e_total // 16
    nchunks = per_tile // k
    base0 = s * per_tile
    semi = [semi0, semi1]
    semg = [semg0, semg1]
    sems = [sems0, sems1]
    mv = m_v[...]

    def fire_idx(g, b):
        pltpu.async_copy(edges_hbm.at[:, pl.ds(base0 + g * k, k)],
                         ebuf.at[b], semi[b])

    def fire_gathers(g, b):
        for j in range(k // 16):
            sv = ebuf[b, 0, pl.ds(j * 16, 16)]
            srcadj[b, pl.ds(j * 16, 16)] = sv + coff
        pltpu.async_copy(pack_hbm.at[srcadj.at[b]], rows2.at[b], semg[b])
        pltpu.async_copy(sd_hbm.at[ebuf.at[b, 1]], sdrows2.at[b], semg[b])

    def consume(g, b):
        for j in range(k // 16):
            ridx = iota + j * 16
            ss = plsc.load_gather(rows2.at[b], [ridx, jnp.full((16,), 26, _i32)])
            sd = plsc.load_gather(sdrows2.at[b], [ridx, jnp.full((16,), 0, _i32)])
            al = ss + sd
            al = jnp.where(al >= 0.0, al, 0.2 * al)
            ev = jnp.exp(al - mv)
            dbuf[b, pl.ds(j * 16, 16)] = ebuf[b, 1, pl.ds(j * 16, 16)]
            for i in range(16):
                row = j * 16 + i
                es = jnp.take(ev, jnp.full((16,), i, _i32))
                scaled2[b, row, pl.ds(0, 16)] = rows2[b, row, pl.ds(0, 16)] * es
                scaled2[b, row, pl.ds(16, 16)] = rows2[b, row, pl.ds(16, 16)] * es
        pltpu.async_copy(scaled2.at[b], acc_sh.at[dbuf.at[b]], sems[b],
                         add=True)

    def drain_scatter(b):
        pltpu.make_async_copy(scaled2.at[b], acc_sh.at[dbuf.at[b]],
                              sems[b]).wait()

    fire_idx(0, 0)
    pltpu.make_async_copy(edges_hbm.at[:, pl.ds(0, k)], ebuf.at[0],
                          semi[0]).wait()
    fire_gathers(0, 0)
    fire_idx(1, 1)

    def pair(g2, _):
        for b in range(2):
            nb = 1 - b
            g = g2 * 2 + b

            @pl.when(g + 1 < nchunks)
            def _():
                pltpu.make_async_copy(
                    edges_hbm.at[:, pl.ds(0, k)], ebuf.at[nb], semi[nb]).wait()
                fire_gathers(g + 1, nb)

            @pl.when(g >= 2)
            def _():
                drain_scatter(b)
            pltpu.make_async_copy(pack_hbm.at[srcadj.at[b]], rows2.at[b],
                                  semg[b]).wait()
            pltpu.make_async_copy(sd_hbm.at[ebuf.at[b, 1]], sdrows2.at[b],
                                  semg[b]).wait()
            consume(g, b)

            @pl.when(g + 2 < nchunks)
            def _():
                fire_idx(g + 2, b)
        return 0

    lax.fori_loop(0, nchunks // 2, pair, 0)
    drain_scatter(0)
    drain_scatter(1)
    plsc.subcore_barrier()
    pltpu.sync_copy(acc_sh.at[pl.ds(s * rows_per_tile, rows_per_tile)],
                    out_hbm.at[pl.ds(coff + s * rows_per_tile, rows_per_tile)])


def _sc_edge_pass(pack_flat, sd16, edge_index, m_vec, zeros_nk):
    n = sd16.shape[0]
    mesh = plsc.VectorSubcoreMesh(core_axis_name="c", subcore_axis_name="s")
    kern = functools.partial(
        pl.kernel,
        mesh=mesh,
        out_type=jax.ShapeDtypeStruct((2 * n, PACKW), _f32),
        scratch_types=[
            pltpu.MemorySpace.VMEM((2, 2, _EK), _i32),
            pltpu.MemorySpace.VMEM((2, _EK), _i32),
            pltpu.MemorySpace.VMEM((2, _EK, PACKW), _f32),
            pltpu.MemorySpace.VMEM((2, _EK, 16), _f32),
            pltpu.MemorySpace.VMEM((2, _EK, PACKW), _f32),
            pltpu.MemorySpace.VMEM((2, _EK), _i32),
            pltpu.MemorySpace.VMEM((16,), _f32),
            pltpu.SemaphoreType.DMA,
            pltpu.SemaphoreType.DMA,
            pltpu.SemaphoreType.DMA,
            pltpu.SemaphoreType.DMA,
            pltpu.SemaphoreType.DMA,
            pltpu.SemaphoreType.DMA,
            pltpu.MemorySpace.VMEM_SHARED((n, PACKW), _f32),
        ],
        compiler_params=pltpu.CompilerParams(needs_layout_passes=False, use_tc_tiling_on_sc=False),
    )(_edge_body)
    return kern(pack_flat, sd16, edge_index, m_vec, zeros_nk)


def _aug_w(w_part, a_src, a_dst):
    k = w_part.shape[0]
    return jnp.concatenate(
        [w_part, (w_part @ a_src)[:, None], (w_part @ a_dst)[:, None],
         jnp.zeros((k, DP - D - 2), _f32)], axis=1)


def kernel(h_attributes, r_idx, t_attributes, edge_index, attr_table, rel_table,
           femb_W, femb_b, gat1_W, gat1_att_src, gat1_att_dst, gat1_bias,
           gat2_W, gat2_att_src, gat2_att_dst, gat2_bias):
    n = h_attributes.shape[0]

    wp = jnp.concatenate([femb_W, jnp.zeros((D, DP - D), _f32)], axis=1)
    bp = jnp.concatenate([femb_b, jnp.zeros((DP - D,), _f32)])[None, :]
    w1_mid_aug = _aug_w(gat1_W[D:2 * D], gat1_att_src, gat1_att_dst)
    w_cat1 = jnp.zeros((_XW, DP), _f32)
    w_cat1 = w_cat1.at[0:D].set(_aug_w(gat1_W[0:D], gat1_att_src, gat1_att_dst))
    w_cat1 = w_cat1.at[DP:DP + D].set(
        _aug_w(gat1_W[2 * D:3 * D], gat1_att_src, gat1_att_dst))
    w2_aug = jnp.zeros((DP, DP), _f32)
    w2_aug = w2_aug.at[0:D].set(_aug_w(gat2_W, gat2_att_src, gat2_att_dst))

    eye = jnp.eye(DP, dtype=_f32)
    p0 = jnp.zeros((DP, PACKW), _f32).at[0:HALF, 0:HALF].set(jnp.eye(HALF, dtype=_f32))
    p1 = jnp.zeros((DP, PACKW), _f32).at[HALF:2 * HALF, 0:HALF].set(
        jnp.eye(HALF, dtype=_f32))
    p0 = p0.at[D, 26].set(1.0)
    p1 = p1.at[D, 26].set(1.0)
    psd = jnp.zeros((DP, 16), _f32).at[D + 1, 0].set(1.0)
    u_row = jnp.zeros((1, PACKW), _f32).at[0, ONES_COL].set(1.0)
    ps = jnp.zeros((DP, 2), _f32).at[D, 0].set(1.0).at[D + 1, 1].set(1.0)
    q0 = jnp.zeros((PACKW, DP), _f32).at[0:HALF, 0:HALF].set(jnp.eye(HALF, dtype=_f32))
    q1 = jnp.zeros((PACKW, DP), _f32).at[0:HALF, HALF:2 * HALF].set(
        jnp.eye(HALF, dtype=_f32))
    qd = jnp.zeros((PACKW, DP), _f32).at[ONES_COL, :].set(1.0)
    s_mat = eye.at[D:, :].set(0.0)
    b_mat = jnp.zeros((DP, DP), _f32).at[D, :].set(1.0).at[D + 1, :].set(1.0)
    bias1_row = jnp.concatenate([gat1_bias, jnp.zeros((DP - D,), _f32)])[None, :]
    bias2_row = jnp.concatenate([gat2_bias, jnp.zeros((DP - D,), _f32)])[None, :]

    hidx = h_attributes.reshape(-1).astype(_i32)
    tidx = t_attributes.reshape(-1).astype(_i32)
    ridx = r_idx.astype(_i32)
    edges = edge_index.astype(_i32)
    zeros_nk = jnp.zeros((n, PACKW), _f32)

    proj_attr = _tc_project_attr(attr_table, wp, bp)
    proj_rel, rel_aug1 = _tc_rel_tables(rel_table, wp, bp, w1_mid_aug)

    x_sc, x_rel = _sc_stage_e(proj_attr, proj_rel, rel_aug1, hidx, tidx, ridx)

    def gat_layer(x, w_aug, bias_row, x_rel=None):
        h_aug, s2, pack, sd16 = _tc_pre(x, w_aug, p0, p1, ps, u_row, psd, x_rel)
        mx = jnp.max(s2, axis=0)
        m = mx[0] + mx[1]
        m = jnp.where(m >= 0.0, m, 0.2 * m)
        m_vec = jnp.full((16,), m, _f32)
        m_row = jnp.full((1, DP), m, _f32)
        acc = _sc_edge_pass(pack.reshape(2 * n, PACKW), sd16, edges,
                            m_vec, zeros_nk)
        return _tc_combine(acc.reshape(2, n, PACKW), h_aug, m_row, bias_row,
                           q0, q1, qd, s_mat, b_mat)

    x1 = gat_layer(x_sc, w_cat1, bias1_row, x_rel)
    x2 = gat_layer(x1, w2_aug, bias2_row)
    return x2[:, :D]


# speedup vs baseline: 2.0447x; 1.0125x over previous
import functools

import jax
import jax.numpy as jnp
from jax import lax
from jax.experimental import pallas as pl
from jax.experimental.pallas import tpu as pltpu
from jax.experimental.pallas import tpu_sc as plsc

_f32 = jnp.float32
_i32 = jnp.int32

D = 50
DP = 64
A = 20
HALF = 25
PACKW = 32
ONES_COL = HALF


def _proj_attr_body(x_ref, w_ref, b_ref, o_ref):
    o_ref[...] = (
        jnp.dot(x_ref[...], w_ref[...], preferred_element_type=_f32) + b_ref[...]
    )


def _tc_project_attr(attr_table, wp, bp):
    v = attr_table.shape[0]
    bl = 2000
    return pl.pallas_call(
        _proj_attr_body,
        grid=(v // bl,),
        in_specs=[
            pl.BlockSpec((bl, attr_table.shape[1]), lambda i: (i, 0)),
            pl.BlockSpec((wp.shape[0], DP), lambda i: (0, 0)),
            pl.BlockSpec((1, DP), lambda i: (0, 0)),
        ],
        out_specs=pl.BlockSpec((bl, DP), lambda i: (i, 0)),
        out_shape=jax.ShapeDtypeStruct((v, DP), _f32),
    )(attr_table, wp, bp)


def _rel_body(r_ref, wp_ref, bp_ref, waug_ref, oproj_ref, oaug_ref):
    r = r_ref[...]
    oproj_ref[...] = jnp.dot(r, wp_ref[...], preferred_element_type=_f32) + bp_ref[...]
    oaug_ref[...] = jnp.dot(r, waug_ref[...], preferred_element_type=_f32)


def _tc_rel_tables(rel_table, wp, bp, w_mid_aug):
    nr = rel_table.shape[0]
    return pl.pallas_call(
        _rel_body,
        out_shape=(
            jax.ShapeDtypeStruct((nr, DP), _f32),
            jax.ShapeDtypeStruct((nr, DP), _f32),
        ),
    )(rel_table, wp, bp, w_mid_aug)


def _pre_finish(h, ps_ref, u_ref, p0_ref, p1_ref, psd_ref, h_ref, s2_ref,
                pack_ref, sd_ref):
    h_ref[...] = h
    s2_ref[...] = jnp.dot(h, ps_ref[...], preferred_element_type=_f32)
    pack_ref[0] = jnp.dot(h, p0_ref[...], preferred_element_type=_f32) + u_ref[...]
    pack_ref[1] = jnp.dot(h, p1_ref[...], preferred_element_type=_f32) + u_ref[...]
    sd_ref[...] = jnp.dot(h, psd_ref[...], preferred_element_type=_f32)


def _pre_body(x_ref, w_ref, p0_ref, p1_ref, ps_ref, u_ref, psd_ref, h_ref,
              s2_ref, pack_ref, sd_ref):
    h = jnp.dot(x_ref[...], w_ref[...], preferred_element_type=_f32)
    _pre_finish(h, ps_ref, u_ref, p0_ref, p1_ref, psd_ref, h_ref, s2_ref,
                pack_ref, sd_ref)


def _pre_body_rel(x_ref, xr_ref, w_ref, p0_ref, p1_ref, ps_ref, u_ref, psd_ref,
                  h_ref, s2_ref, pack_ref, sd_ref):
    h = jnp.dot(x_ref[...], w_ref[...], preferred_element_type=_f32) + xr_ref[...]
    _pre_finish(h, ps_ref, u_ref, p0_ref, p1_ref, psd_ref, h_ref, s2_ref,
                pack_ref, sd_ref)


def _tc_pre(x, w_aug, p0, p1, ps, u_row, psd, x_rel=None):
    n = x.shape[0]
    xw = x.shape[1]
    bl = 1000
    xspec = [pl.BlockSpec((bl, xw), lambda i: (i, 0))]
    args = [x]
    body = _pre_body
    if x_rel is not None:
        xspec.append(pl.BlockSpec((bl, DP), lambda i: (i, 0)))
        args.append(x_rel)
        body = _pre_body_rel
    return pl.pallas_call(
        body,
        grid=(n // bl,),
        in_specs=xspec + [
            pl.BlockSpec((xw, DP), lambda i: (0, 0)),
            pl.BlockSpec((DP, PACKW), lambda i: (0, 0)),
            pl.BlockSpec((DP, PACKW), lambda i: (0, 0)),
            pl.BlockSpec((DP, 2), lambda i: (0, 0)),
            pl.BlockSpec((1, PACKW), lambda i: (0, 0)),
            pl.BlockSpec((DP, 16), lambda i: (0, 0)),
        ],
        out_specs=(
            pl.BlockSpec((bl, DP), lambda i: (i, 0)),
            pl.BlockSpec((bl, 2), lambda i: (i, 0)),
            pl.BlockSpec((2, bl, PACKW), lambda i: (0, i, 0)),
            pl.BlockSpec((bl, 16), lambda i: (i, 0)),
        ),
        out_shape=(
            jax.ShapeDtypeStruct((n, DP), _f32),
            jax.ShapeDtypeStruct((n, 2), _f32),
            jax.ShapeDtypeStruct((2, n, PACKW), _f32),
            jax.ShapeDtypeStruct((n, 16), _f32),
        ),
    )(*args, w_aug, p0, p1, ps, u_row, psd)


def _combine_body(acc_ref, h_ref, m_ref, b_ref, q0_ref, q1_ref, qd_ref, s_ref,
                  bb_ref, o_ref):
    a0 = acc_ref[0]
    a1 = acc_ref[1]
    h = h_ref[...]
    num = (jnp.dot(a0, q0_ref[...], preferred_element_type=_f32)
           + jnp.dot(a1, q1_ref[...], preferred_element_type=_f32))
    den = jnp.dot(a0, qd_ref[...], preferred_element_type=_f32)
    h_sel = jnp.dot(h, s_ref[...], preferred_element_type=_f32)
    ssum = jnp.dot(h, bb_ref[...], preferred_element_type=_f32)
    ssum = jnp.where(ssum >= 0.0, ssum, 0.2 * ssum)
    el = jnp.exp(ssum - m_ref[...])
    o_ref[...] = (num + el * h_sel) / (den + el + 1e-16) + b_ref[...]


def _combine_pre_body(acc_ref, h_ref, m_ref, b_ref, q0_ref, q1_ref, qd_ref,
                      s_ref, bb_ref, w_ref, p0_ref, p1_ref, ps_ref, u_ref,
                      psd_ref, h2_ref, s2_ref, pack_ref, sd_ref):
    a0 = acc_ref[0]
    a1 = acc_ref[1]
    h = h_ref[...]
    num = (jnp.dot(a0, q0_ref[...], preferred_element_type=_f32)
           + jnp.dot(a1, q1_ref[...], preferred_element_type=_f32))
    den = jnp.dot(a0, qd_ref[...], preferred_element_type=_f32)
    h_sel = jnp.dot(h, s_ref[...], preferred_element_type=_f32)
    ssum = jnp.dot(h, bb_ref[...], preferred_element_type=_f32)
    ssum = jnp.where(ssum >= 0.0, ssum, 0.2 * ssum)
    el = jnp.exp(ssum - m_ref[...])
    x2 = (num + el * h_sel) / (den + el + 1e-16) + b_ref[...]
    h2 = jnp.dot(x2, w_ref[...], preferred_element_type=_f32)
    _pre_finish(h2, ps_ref, u_ref, p0_ref, p1_ref, psd_ref, h2_ref, s2_ref,
                pack_ref, sd_ref)


def _tc_combine_pre(acc, h_aug, m_row, bias_row, q0, q1, qd, s_mat, b_mat,
                    w_aug, p0, p1, ps, u_row, psd):
    n = h_aug.shape[0]
    bl = 1000
    full = lambda shape: pl.BlockSpec(shape, lambda i: tuple(0 for _ in shape))
    return pl.pallas_call(
        _combine_pre_body,
        grid=(n // bl,),
        in_specs=[
            pl.BlockSpec((2, bl, PACKW), lambda i: (0, i, 0)),
            pl.BlockSpec((bl, DP), lambda i: (i, 0)),
            full((1, DP)), full((1, DP)),
            full((PACKW, DP)), full((PACKW, DP)), full((PACKW, DP)),
            full((DP, DP)), full((DP, DP)),
            full((DP, DP)),
            full((DP, PACKW)), full((DP, PACKW)), full((DP, 2)),
            full((1, PACKW)), full((DP, 16)),
        ],
        out_specs=(
            pl.BlockSpec((bl, DP), lambda i: (i, 0)),
            pl.BlockSpec((bl, 2), lambda i: (i, 0)),
            pl.BlockSpec((2, bl, PACKW), lambda i: (0, i, 0)),
            pl.BlockSpec((bl, 16), lambda i: (i, 0)),
        ),
        out_shape=(
            jax.ShapeDtypeStruct((n, DP), _f32),
            jax.ShapeDtypeStruct((n, 2), _f32),
            jax.ShapeDtypeStruct((2, n, PACKW), _f32),
            jax.ShapeDtypeStruct((n, 16), _f32),
        ),
    )(acc, h_aug, m_row, bias_row, q0, q1, qd, s_mat, b_mat,
      w_aug, p0, p1, ps, u_row, psd)


def _tc_combine(acc, h_aug, m_row, bias_row, q0, q1, qd, s_mat, b_mat):
    n = h_aug.shape[0]
    bl = 1000
    return pl.pallas_call(
        _combine_body,
        grid=(n // bl,),
        in_specs=[
            pl.BlockSpec((2, bl, PACKW), lambda i: (0, i, 0)),
            pl.BlockSpec((bl, DP), lambda i: (i, 0)),
            pl.BlockSpec((1, DP), lambda i: (0, 0)),
            pl.BlockSpec((1, DP), lambda i: (0, 0)),
            pl.BlockSpec((PACKW, DP), lambda i: (0, 0)),
            pl.BlockSpec((PACKW, DP), lambda i: (0, 0)),
            pl.BlockSpec((PACKW, DP), lambda i: (0, 0)),
            pl.BlockSpec((DP, DP), lambda i: (0, 0)),
            pl.BlockSpec((DP, DP), lambda i: (0, 0)),
        ],
        out_specs=pl.BlockSpec((bl, DP), lambda i: (i, 0)),
        out_shape=jax.ShapeDtypeStruct((n, DP), _f32),
    )(acc, h_aug, m_row, bias_row, q0, q1, qd, s_mat, b_mat)


_XW = 128


def _stage_e_body(proj_attr, proj_rel, rel_aug, hidx_hbm, tidx_hbm, ridx_hbm,
                  x_hbm, xrel_hbm,
                  hib, tib, rib, ah2, at2, relp2, relaug2, xrow,
                  semi0, semi1, semg0, semg1):
    c = lax.axis_index("c")
    s = lax.axis_index("s")
    w = s * 2 + c
    iota = lax.broadcasted_iota(_i32, (16,), 0)
    c20 = iota * A
    zero16 = jnp.zeros((16,), _f32)
    semi = [semi0, semi1]
    semg = [semg0, semg1]

    for r in range(16):
        for k in range(_XW // 16):
            xrow[r, pl.ds(k * 16, 16)] = zero16

    nblocks = 3125
    count = (nblocks - w + 31) // 32

    def node0_of(g):
        return (w + g * 32) * 16

    def fire_idx(g, b):
        n0 = node0_of(g)
        pltpu.async_copy(hidx_hbm.at[pl.ds(n0 * A, 16 * A)], hib.at[b],
                         semi[b])
        pltpu.async_copy(tidx_hbm.at[pl.ds(n0 * A, 16 * A)], tib.at[b],
                         semi[b])
        pltpu.async_copy(ridx_hbm.at[pl.ds(n0, 16)], rib.at[b], semi[b])

    def wait_idx(b):
        pltpu.make_async_copy(hidx_hbm.at[pl.ds(0, 16 * A)], hib.at[b],
                              semi[b]).wait()
        pltpu.make_async_copy(tidx_hbm.at[pl.ds(0, 16 * A)], tib.at[b],
                              semi[b]).wait()
        pltpu.make_async_copy(ridx_hbm.at[pl.ds(0, 16)], rib.at[b],
                              semi[b]).wait()

    def fire_gathers(b):
        for k in range(4):
            pltpu.async_copy(proj_attr.at[hib.at[b, pl.ds(k * 80, 80)]],
                             ah2.at[b, pl.ds(k * 80, 80)], semg[b])
            pltpu.async_copy(proj_attr.at[tib.at[b, pl.ds(k * 80, 80)]],
                             at2.at[b, pl.ds(k * 80, 80)], semg[b])
        pltpu.async_copy(proj_rel.at[rib.at[b]], relp2.at[b], semg[b])
        pltpu.async_copy(rel_aug.at[rib.at[b]], relaug2.at[b], semg[b])

    def wait_gathers(b):
        for k in range(4):
            pltpu.make_async_copy(proj_attr.at[hib.at[b, pl.ds(k * 80, 80)]],
                                  ah2.at[b, pl.ds(k * 80, 80)], semg[b]).wait()
            pltpu.make_async_copy(proj_attr.at[tib.at[b, pl.ds(k * 80, 80)]],
                                  at2.at[b, pl.ds(k * 80, 80)], semg[b]).wait()
        pltpu.make_async_copy(proj_rel.at[rib.at[b]], relp2.at[b],
                              semg[b]).wait()
        pltpu.make_async_copy(rel_aug.at[rib.at[b]], relaug2.at[b],
                              semg[b]).wait()

    neg_big = jnp.full((16,), -1e30, _f32)
    nchunk = DP // 16

    def attend(av_ref, rp_ref, off, j):
        rv = [rp_ref[j, pl.ds(kk * 16, 16)] for kk in range(nchunk)]
        lv0 = neg_big
        lv1 = neg_big
        for a in range(A):
            row = j * A + a
            part = av_ref[row, pl.ds(0, 16)] * rv[0]
            for kk in range(1, nchunk):
                part = part + av_ref[row, pl.ds(kk * 16, 16)] * rv[kk]
            tot = jnp.sum(part)
            if a < 16:
                lv0 = jnp.where(iota == a, tot, lv0)
            else:
                lv1 = jnp.where(iota == (a - 16), tot, lv1)
        m = jnp.maximum(jnp.max(lv0), jnp.max(lv1))
        e0 = jnp.exp(lv0 - m)
        e1 = jnp.exp(lv1 - m)
        tot = jnp.sum(e0) + jnp.sum(e1)
        attn0 = e0 / tot
        attn1 = e1 / tot
        hv = [zero16 for _ in range(nchunk)]
        for a in range(A):
            row = j * A + a
            aa = attn0[a] if a < 16 else attn1[a - 16]
            for kk in range(nchunk):
                hv[kk] = hv[kk] + av_ref[row, pl.ds(kk * 16, 16)] * aa
        for kk in range(nchunk):
            xrow[j, pl.ds(off + kk * 16, 16)] = hv[kk]

    fire_idx(0, 0)
    wait_idx(0)
    fire_gathers(0)
    fire_idx(1, 1)

    def pair(g2, _):
        for b in range(2):
            nb = 1 - b
            g = g2 * 2 + b

            @pl.when(g < count)
            def _():
                @pl.when(g + 1 < count)
                def _():
                    wait_idx(nb)
                    fire_gathers(nb)
                wait_gathers(b)

                def node_step(j, _):
                    attend(ah2.at[b], relp2.at[b], 0, j)
                    attend(at2.at[b], relp2.at[b], DP, j)
                    return 0

                lax.fori_loop(0, 16, node_step, 0)
                n0 = node0_of(g)
                pltpu.sync_copy(xrow, x_hbm.at[pl.ds(n0, 16)])
                pltpu.sync_copy(relaug2.at[b], xrel_hbm.at[pl.ds(n0, 16)])

                @pl.when(g + 2 < count)
                def _():
                    fire_idx(g + 2, b)
        return 0

    lax.fori_loop(0, (count + 1) // 2, pair, 0)


def _sc_stage_e(proj_attr, proj_rel, rel_aug, hidx, tidx, ridx):
    n = ridx.shape[0]
    mesh = plsc.VectorSubcoreMesh(core_axis_name="c", subcore_axis_name="s")
    kern = functools.partial(
        pl.kernel,
        mesh=mesh,
        out_type=(
            jax.ShapeDtypeStruct((n, _XW), _f32),
            jax.ShapeDtypeStruct((n, DP), _f32),
        ),
        scratch_types=[
            pltpu.MemorySpace.VMEM((2, 320), _i32),
            pltpu.MemorySpace.VMEM((2, 320), _i32),
            pltpu.MemorySpace.VMEM((2, 16), _i32),
            pltpu.MemorySpace.VMEM((2, 320, DP), _f32),
            pltpu.MemorySpace.VMEM((2, 320, DP), _f32),
            pltpu.MemorySpace.VMEM((2, 16, DP), _f32),
            pltpu.MemorySpace.VMEM((2, 16, DP), _f32),
            pltpu.MemorySpace.VMEM((16, _XW), _f32),
            pltpu.SemaphoreType.DMA,
            pltpu.SemaphoreType.DMA,
            pltpu.SemaphoreType.DMA,
            pltpu.SemaphoreType.DMA,
        ],
        compiler_params=pltpu.CompilerParams(needs_layout_passes=False, use_tc_tiling_on_sc=False),
    )(_stage_e_body)
    return kern(proj_attr, proj_rel, rel_aug, hidx, tidx, ridx)


SSRC_LANE = 10


_EK = 80


def _edge_body(pack_hbm, sd_hbm, edges_hbm, m_hbm, zeros_hbm, out_hbm,
               ebuf, srcadj, rows2, sdrows2, scaled2, dbuf, m_v,
               semi0, semi1, semg0, semg1, sems0, sems1, acc_sh):
    c = lax.axis_index("c")
    s = lax.axis_index("s")
    n = sd_hbm.shape[0]
    rows_per_tile = n // 16
    coff = c * n
    k = _EK
    iota = lax.broadcasted_iota(_i32, (16,), 0)

    pltpu.sync_copy(m_hbm, m_v)
    pltpu.sync_copy(zeros_hbm.at[pl.ds(s * rows_per_tile, rows_per_tile)],
                    acc_sh.at[pl.ds(s * rows_per_tile, rows_per_tile)])
    plsc.subcore_barrier()

    e_total = edges_hbm.shape[1]
    per_tile = e_total // 16
    nchunks = per_tile // k
    base0 = s * per_tile
    semi = [semi0, semi1]
    semg = [semg0, semg1]
    sems = [sems0, sems1]
    mv = m_v[...]

    def fire_idx(g, b):
        pltpu.async_copy(edges_hbm.at[:, pl.ds(base0 + g * k, k)],
                         ebuf.at[b], semi[b])

    def fire_gathers(g, b):
        for j in range(k // 16):
            sv = ebuf[b, 0, pl.ds(j * 16, 16)]
            srcadj[b, pl.ds(j * 16, 16)] = sv + coff
        pltpu.async_copy(pack_hbm.at[srcadj.at[b]], rows2.at[b], semg[b])
        pltpu.async_copy(sd_hbm.at[ebuf.at[b, 1]], sdrows2.at[b], semg[b])

    def consume(g, b):
        for j in range(k // 16):
            ridx = iota + j * 16
            ss = plsc.load_gather(rows2.at[b], [ridx, jnp.full((16,), 26, _i32)])
            sd = plsc.load_gather(sdrows2.at[b], [ridx, jnp.full((16,), 0, _i32)])
            al = ss + sd
            al = jnp.where(al >= 0.0, al, 0.2 * al)
            ev = jnp.exp(al - mv)
            dbuf[b, pl.ds(j * 16, 16)] = ebuf[b, 1, pl.ds(j * 16, 16)]
            for i in range(16):
                row = j * 16 + i
                es = jnp.take(ev, jnp.full((16,), i, _i32))
                scaled2[b, row, pl.ds(0, 16)] = rows2[b, row, pl.ds(0, 16)] * es
                scaled2[b, row, pl.ds(16, 16)] = rows2[b, row, pl.ds(16, 16)] * es
        pltpu.async_copy(scaled2.at[b], acc_sh.at[dbuf.at[b]], sems[b],
                         add=True)

    def drain_scatter(b):
        pltpu.make_async_copy(scaled2.at[b], acc_sh.at[dbuf.at[b]],
                              sems[b]).wait()

    fire_idx(0, 0)
    pltpu.make_async_copy(edges_hbm.at[:, pl.ds(0, k)], ebuf.at[0],
                          semi[0]).wait()
    fire_gathers(0, 0)
    fire_idx(1, 1)

    def pair(g2, _):
        for b in range(2):
            nb = 1 - b
            g = g2 * 2 + b

            @pl.when(g + 1 < nchunks)
            def _():
                pltpu.make_async_copy(
                    edges_hbm.at[:, pl.ds(0, k)], ebuf.at[nb], semi[nb]).wait()
                fire_gathers(g + 1, nb)

            @pl.when(g >= 2)
            def _():
                drain_scatter(b)
            pltpu.make_async_copy(pack_hbm.at[srcadj.at[b]], rows2.at[b],
                                  semg[b]).wait()
            pltpu.make_async_copy(sd_hbm.at[ebuf.at[b, 1]], sdrows2.at[b],
                                  semg[b]).wait()
            consume(g, b)

            @pl.when(g + 2 < nchunks)
            def _():
                fire_idx(g + 2, b)
        return 0

    lax.fori_loop(0, nchunks // 2, pair, 0)
    drain_scatter(0)
    drain_scatter(1)
    plsc.subcore_barrier()
    pltpu.sync_copy(acc_sh.at[pl.ds(s * rows_per_tile, rows_per_tile)],
                    out_hbm.at[pl.ds(coff + s * rows_per_tile, rows_per_tile)])


def _sc_edge_pass(pack_flat, sd16, edge_index, m_vec, zeros_nk):
    n = sd16.shape[0]
    mesh = plsc.VectorSubcoreMesh(core_axis_name="c", subcore_axis_name="s")
    kern = functools.partial(
        pl.kernel,
        mesh=mesh,
        out_type=jax.ShapeDtypeStruct((2 * n, PACKW), _f32),
        scratch_types=[
            pltpu.MemorySpace.VMEM((2, 2, _EK), _i32),
            pltpu.MemorySpace.VMEM((2, _EK), _i32),
            pltpu.MemorySpace.VMEM((2, _EK, PACKW), _f32),
            pltpu.MemorySpace.VMEM((2, _EK, 16), _f32),
            pltpu.MemorySpace.VMEM((2, _EK, PACKW), _f32),
            pltpu.MemorySpace.VMEM((2, _EK), _i32),
            pltpu.MemorySpace.VMEM((16,), _f32),
            pltpu.SemaphoreType.DMA,
            pltpu.SemaphoreType.DMA,
            pltpu.SemaphoreType.DMA,
            pltpu.SemaphoreType.DMA,
            pltpu.SemaphoreType.DMA,
            pltpu.SemaphoreType.DMA,
            pltpu.MemorySpace.VMEM_SHARED((n, PACKW), _f32),
        ],
        compiler_params=pltpu.CompilerParams(needs_layout_passes=False, use_tc_tiling_on_sc=False),
    )(_edge_body)
    return kern(pack_flat, sd16, edge_index, m_vec, zeros_nk)


def _aug_w(w_part, a_src, a_dst):
    k = w_part.shape[0]
    return jnp.concatenate(
        [w_part, (w_part @ a_src)[:, None], (w_part @ a_dst)[:, None],
         jnp.zeros((k, DP - D - 2), _f32)], axis=1)


def kernel(h_attributes, r_idx, t_attributes, edge_index, attr_table, rel_table,
           femb_W, femb_b, gat1_W, gat1_att_src, gat1_att_dst, gat1_bias,
           gat2_W, gat2_att_src, gat2_att_dst, gat2_bias):
    n = h_attributes.shape[0]

    wp = jnp.concatenate([femb_W, jnp.zeros((D, DP - D), _f32)], axis=1)
    bp = jnp.concatenate([femb_b, jnp.zeros((DP - D,), _f32)])[None, :]
    w1_mid_aug = _aug_w(gat1_W[D:2 * D], gat1_att_src, gat1_att_dst)
    w_cat1 = jnp.zeros((_XW, DP), _f32)
    w_cat1 = w_cat1.at[0:D].set(_aug_w(gat1_W[0:D], gat1_att_src, gat1_att_dst))
    w_cat1 = w_cat1.at[DP:DP + D].set(
        _aug_w(gat1_W[2 * D:3 * D], gat1_att_src, gat1_att_dst))
    w2_aug = jnp.zeros((DP, DP), _f32)
    w2_aug = w2_aug.at[0:D].set(_aug_w(gat2_W, gat2_att_src, gat2_att_dst))

    eye = jnp.eye(DP, dtype=_f32)
    p0 = jnp.zeros((DP, PACKW), _f32).at[0:HALF, 0:HALF].set(jnp.eye(HALF, dtype=_f32))
    p1 = jnp.zeros((DP, PACKW), _f32).at[HALF:2 * HALF, 0:HALF].set(
        jnp.eye(HALF, dtype=_f32))
    p0 = p0.at[D, 26].set(1.0)
    p1 = p1.at[D, 26].set(1.0)
    psd = jnp.zeros((DP, 16), _f32).at[D + 1, 0].set(1.0)
    u_row = jnp.zeros((1, PACKW), _f32).at[0, ONES_COL].set(1.0)
    ps = jnp.zeros((DP, 2), _f32).at[D, 0].set(1.0).at[D + 1, 1].set(1.0)
    q0 = jnp.zeros((PACKW, DP), _f32).at[0:HALF, 0:HALF].set(jnp.eye(HALF, dtype=_f32))
    q1 = jnp.zeros((PACKW, DP), _f32).at[0:HALF, HALF:2 * HALF].set(
        jnp.eye(HALF, dtype=_f32))
    qd = jnp.zeros((PACKW, DP), _f32).at[ONES_COL, :].set(1.0)
    s_mat = eye.at[D:, :].set(0.0)
    b_mat = jnp.zeros((DP, DP), _f32).at[D, :].set(1.0).at[D + 1, :].set(1.0)
    bias1_row = jnp.concatenate([gat1_bias, jnp.zeros((DP - D,), _f32)])[None, :]
    bias2_row = jnp.concatenate([gat2_bias, jnp.zeros((DP - D,), _f32)])[None, :]

    hidx = h_attributes.reshape(-1).astype(_i32)
    tidx = t_attributes.reshape(-1).astype(_i32)
    ridx = r_idx.astype(_i32)
    edges = edge_index.astype(_i32)
    zeros_nk = jnp.zeros((n, PACKW), _f32)

    proj_attr = _tc_project_attr(attr_table, wp, bp)
    proj_rel, rel_aug1 = _tc_rel_tables(rel_table, wp, bp, w1_mid_aug)

    x_sc, x_rel = _sc_stage_e(proj_attr, proj_rel, rel_aug1, hidx, tidx, ridx)

    def score_stats(s2):
        mx = jnp.max(s2, axis=0)
        m = mx[0] + mx[1]
        m = jnp.where(m >= 0.0, m, 0.2 * m)
        return jnp.full((16,), m, _f32), jnp.full((1, DP), m, _f32)

    h1_aug, s21, pack1, sd1 = _tc_pre(x_sc, w_cat1, p0, p1, ps, u_row, psd,
                                      x_rel)
    m1_vec, m1_row = score_stats(s21)
    acc1 = _sc_edge_pass(pack1.reshape(2 * n, PACKW), sd1, edges, m1_vec,
                         zeros_nk)
    h2_aug, s22, pack2, sd2 = _tc_combine_pre(
        acc1.reshape(2, n, PACKW), h1_aug, m1_row, bias1_row,
        q0, q1, qd, s_mat, b_mat, w2_aug, p0, p1, ps, u_row, psd)
    m2_vec, m2_row = score_stats(s22)
    acc2 = _sc_edge_pass(pack2.reshape(2 * n, PACKW), sd2, edges, m2_vec,
                         zeros_nk)
    out = _tc_combine(acc2.reshape(2, n, PACKW), h2_aug, m2_row, bias2_row,
                      q0, q1, qd, s_mat, b_mat)
    return out[:, :D]
